# baseline (speedup 1.0000x reference)
import functools

import jax
import jax.numpy as jnp
from jax import lax
from jax.experimental import pallas as pl
from jax.experimental.pallas import tpu as pltpu
from jax.experimental.pallas import tpu_sc as plsc

BATCH = 4096
D_IN = 4096
K_RAW = 7000
KP = 7040
CD = 32
NQ = 3
BM = 128
GRID = BATCH // BM

_SC_NC = 2
_SC_NS = 16
_NW = _SC_NC * _SC_NS
_BPW = BATCH // _NW


def _vq_argmin(r, cbt2):
    lane1 = lax.broadcasted_iota(jnp.int32, (1, KP), 1)
    lane2 = lax.broadcasted_iota(jnp.int32, (BM, KP), 1)
    n = jnp.sum(cbt2 * cbt2, axis=0, keepdims=True) * jnp.float32(0.25)
    n = jnp.where(lane1 < K_RAW, n, jnp.float32(1e30))
    c = jnp.sum(r * r, axis=-1, keepdims=True)
    m2 = jnp.dot(r, cbt2, preferred_element_type=jnp.float32)
    dist = (c + m2) + n
    mn = jnp.min(dist, axis=-1, keepdims=True)
    return jnp.min(jnp.where(dist == mn, lane2, KP), axis=-1)


def _acc_loss(loss_ref, i, s):
    li = lax.broadcasted_iota(jnp.int32, (8, 128), 1)
    part = jnp.where(li == 0, s, 0.0)

    @pl.when(i == 0)
    def _init():
        loss_ref[...] = jnp.zeros_like(loss_ref)

    loss_ref[...] += part


def _enc_body(x_ref, w0, b0, w1, b1, w2, b2, cbt2_ref, z_ref, idx_ref):
    x = x_ref[...]
    h = jnp.maximum(jnp.dot(x, w0[...], preferred_element_type=jnp.float32) + b0[...], 0.0)
    h = jnp.maximum(jnp.dot(h, w1[...], preferred_element_type=jnp.float32) + b1[...], 0.0)
    z = jnp.dot(h, w2[...], preferred_element_type=jnp.float32) + b2[...]
    z_ref[...] = z
    idx_ref[...] = _vq_argmin(z, cbt2_ref[...])


def _step_body(r_ref, qv_ref, cbt2_ref, rn_ref, idx_ref, loss_ref):
    r = r_ref[...] - qv_ref[:, :CD]
    rn_ref[...] = r
    idx_ref[...] = _vq_argmin(r, cbt2_ref[...])
    _acc_loss(loss_ref, pl.program_id(0), jnp.sum(r * r))


def _fin_body(z_ref, q0_ref, q1_ref, q2_ref, dw0, db0, dw1, db1, dw2, db2,
              recon_ref, loss_ref):
    z = z_ref[...]
    q0, q1, q2 = q0_ref[:, :CD], q1_ref[:, :CD], q2_ref[:, :CD]
    r1 = z - q0
    r2 = r1 - q1
    r3 = r2 - q2
    qst0 = z + (q0 - z)
    qst1 = r1 + (q1 - r1)
    qst2 = r2 + (q2 - r2)
    zq = (qst0 + qst1) + qst2
    g = jnp.maximum(jnp.dot(zq, dw0[...], preferred_element_type=jnp.float32) + db0[...], 0.0)
    g = jnp.maximum(jnp.dot(g, dw1[...], preferred_element_type=jnp.float32) + db1[...], 0.0)
    recon_ref[...] = jnp.dot(g, dw2[...], preferred_element_type=jnp.float32) + db2[...]
    _acc_loss(loss_ref, pl.program_id(0), jnp.sum(r3 * r3))


def _sc_gather(table, idx):
    mesh = plsc.VectorSubcoreMesh(core_axis_name="c", subcore_axis_name="s")

    @functools.partial(
        pl.kernel,
        out_type=jax.ShapeDtypeStruct((BATCH, 128), jnp.float32),
        mesh=mesh,
        scratch_types=[
            pltpu.VMEM((_BPW,), jnp.int32),
            pltpu.VMEM((_BPW, 128), jnp.float32),
            pltpu.SemaphoreType.DMA,
        ],
    )
    def k(table_hbm, idx_hbm, out_hbm, idx_v, rows_v, sem):
        wid = lax.axis_index("s") * _SC_NC + lax.axis_index("c")
        base = wid * _BPW
        pltpu.sync_copy(idx_hbm.at[pl.ds(base, _BPW)], idx_v)
        pltpu.async_copy(table_hbm.at[idx_v], rows_v, sem).wait()
        pltpu.sync_copy(rows_v, out_hbm.at[pl.ds(base, _BPW)])

    return k(table, idx)


def kernel(x, enc_W0, enc_b0, enc_W1, enc_b1, enc_W2, enc_b2,
           dec_W0, dec_b0, dec_W1, dec_b1, dec_W2, dec_b2, codebooks):
    cb_pad = jnp.pad(codebooks, ((0, 0), (0, KP - K_RAW), (0, 0)))
    cbt2 = jnp.transpose(cb_pad, (0, 2, 1)) * jnp.float32(-2.0)
    cb_sc = jnp.pad(codebooks, ((0, 0), (0, 0), (0, 128 - CD)))
    row = lambda v: v.reshape(1, -1)
    full = lambda a: pl.BlockSpec(a.shape, lambda i: (0,) * a.ndim)
    rows_spec = pl.BlockSpec((BM, D_IN), lambda i: (i, 0))
    small_spec = pl.BlockSpec((BM, CD), lambda i: (i, 0))
    gat_spec = pl.BlockSpec((BM, 128), lambda i: (i, 0))
    idx_spec = pl.BlockSpec((BM,), lambda i: (i,))
    loss_spec = pl.BlockSpec((8, 128), lambda i: (0, 0))
    small_shape = jax.ShapeDtypeStruct((BATCH, CD), jnp.float32)
    idx_shape = jax.ShapeDtypeStruct((BATCH,), jnp.int32)
    loss_shape = jax.ShapeDtypeStruct((8, 128), jnp.float32)

    enc_w = [enc_W0, row(enc_b0), enc_W1, row(enc_b1), enc_W2, row(enc_b2)]
    dec_w = [dec_W0, row(dec_b0), dec_W1, row(dec_b1), dec_W2, row(dec_b2)]

    z, i0 = pl.pallas_call(
        _enc_body, grid=(GRID,),
        in_specs=[rows_spec] + [full(w) for w in enc_w] + [full(cbt2[0])],
        out_specs=[small_spec, idx_spec],
        out_shape=[small_shape, idx_shape],
    )(x, *enc_w, cbt2[0])

    q0 = _sc_gather(cb_sc[0], i0)

    r1, i1, lb0 = pl.pallas_call(
        _step_body, grid=(GRID,),
        in_specs=[small_spec, gat_spec, full(cbt2[1])],
        out_specs=[small_spec, idx_spec, loss_spec],
        out_shape=[small_shape, idx_shape, loss_shape],
    )(z, q0, cbt2[1])

    q1 = _sc_gather(cb_sc[1], i1)

    r2, i2, lb1 = pl.pallas_call(
        _step_body, grid=(GRID,),
        in_specs=[small_spec, gat_spec, full(cbt2[2])],
        out_specs=[small_spec, idx_spec, loss_spec],
        out_shape=[small_shape, idx_shape, loss_shape],
    )(r1, q1, cbt2[2])

    q2 = _sc_gather(cb_sc[2], i2)

    recon, lb2 = pl.pallas_call(
        _fin_body, grid=(GRID,),
        in_specs=[small_spec, gat_spec, gat_spec, gat_spec] + [full(w) for w in dec_w],
        out_specs=[rows_spec, loss_spec],
        out_shape=[jax.ShapeDtypeStruct((BATCH, D_IN), jnp.float32), loss_shape],
    )(z, q0, q1, q2, *dec_w)

    indices = jnp.stack([i0, i1, i2], axis=-1)
    commit_loss = jnp.stack([lb0[0, 0], lb1[0, 0], lb2[0, 0]]) * jnp.float32(1.0 / (BATCH * CD))
    return recon, indices, commit_loss


# speedup vs baseline: 1.0983x; 1.0983x over previous
import functools

import jax
import jax.numpy as jnp
from jax import lax
from jax.experimental import pallas as pl
from jax.experimental.pallas import tpu as pltpu
from jax.experimental.pallas import tpu_sc as plsc

BATCH = 4096
D_IN = 4096
K_RAW = 7000
KP = 7040
CD = 32
NQ = 3
BM = 256
GRID = BATCH // BM

_SC_NC = 2
_SC_NS = 16
_NW = _SC_NC * _SC_NS
_BPW = BATCH // _NW


def _vq_argmin(r, cbt2):
    lane1 = lax.broadcasted_iota(jnp.int32, (1, KP), 1)
    lane2 = lax.broadcasted_iota(jnp.int32, (BM, KP), 1)
    n = jnp.sum(cbt2 * cbt2, axis=0, keepdims=True) * jnp.float32(0.25)
    n = jnp.where(lane1 < K_RAW, n, jnp.float32(1e30))
    c = jnp.sum(r * r, axis=-1, keepdims=True)
    m2 = jnp.dot(r, cbt2, preferred_element_type=jnp.float32)
    dist = (c + m2) + n
    mn = jnp.min(dist, axis=-1, keepdims=True)
    return jnp.min(jnp.where(dist == mn, lane2, KP), axis=-1)


def _acc_loss(loss_ref, i, s):
    li = lax.broadcasted_iota(jnp.int32, (8, 128), 1)
    part = jnp.where(li == 0, s, 0.0)

    @pl.when(i == 0)
    def _init():
        loss_ref[...] = jnp.zeros_like(loss_ref)

    loss_ref[...] += part


def _enc_body(x_ref, w0, b0, w1, b1, w2, b2, cbt2_ref, z_ref, idx_ref):
    x = x_ref[...]
    h = jnp.maximum(jnp.dot(x, w0[...], preferred_element_type=jnp.float32) + b0[...], 0.0)
    h = jnp.maximum(jnp.dot(h, w1[...], preferred_element_type=jnp.float32) + b1[...], 0.0)
    z = jnp.dot(h, w2[...], preferred_element_type=jnp.float32) + b2[...]
    z_ref[...] = z
    idx_ref[...] = _vq_argmin(z, cbt2_ref[...])


def _step_body(r_ref, qv_ref, cbt2_ref, rn_ref, idx_ref, loss_ref):
    r = r_ref[...] - qv_ref[:, :CD]
    rn_ref[...] = r
    idx_ref[...] = _vq_argmin(r, cbt2_ref[...])
    _acc_loss(loss_ref, pl.program_id(0), jnp.sum(r * r))


def _fin_body(z_ref, q0_ref, q1_ref, q2_ref, dw0, db0, dw1, db1, dw2, db2,
              recon_ref, loss_ref):
    z = z_ref[...]
    q0, q1, q2 = q0_ref[:, :CD], q1_ref[:, :CD], q2_ref[:, :CD]
    r1 = z - q0
    r2 = r1 - q1
    r3 = r2 - q2
    qst0 = z + (q0 - z)
    qst1 = r1 + (q1 - r1)
    qst2 = r2 + (q2 - r2)
    zq = (qst0 + qst1) + qst2
    g = jnp.maximum(jnp.dot(zq, dw0[...], preferred_element_type=jnp.float32) + db0[...], 0.0)
    g = jnp.maximum(jnp.dot(g, dw1[...], preferred_element_type=jnp.float32) + db1[...], 0.0)
    recon_ref[...] = jnp.dot(g, dw2[...], preferred_element_type=jnp.float32) + db2[...]
    _acc_loss(loss_ref, pl.program_id(0), jnp.sum(r3 * r3))


def _sc_gather(table, idx):
    mesh = plsc.VectorSubcoreMesh(core_axis_name="c", subcore_axis_name="s")

    @functools.partial(
        pl.kernel,
        out_type=jax.ShapeDtypeStruct((BATCH, 128), jnp.float32),
        mesh=mesh,
        scratch_types=[
            pltpu.VMEM((_BPW,), jnp.int32),
            pltpu.VMEM((_BPW, 128), jnp.float32),
            pltpu.SemaphoreType.DMA,
        ],
    )
    def k(table_hbm, idx_hbm, out_hbm, idx_v, rows_v, sem):
        wid = lax.axis_index("s") * _SC_NC + lax.axis_index("c")
        base = wid * _BPW
        pltpu.sync_copy(idx_hbm.at[pl.ds(base, _BPW)], idx_v)
        pltpu.async_copy(table_hbm.at[idx_v], rows_v, sem).wait()
        pltpu.sync_copy(rows_v, out_hbm.at[pl.ds(base, _BPW)])

    return k(table, idx)


def kernel(x, enc_W0, enc_b0, enc_W1, enc_b1, enc_W2, enc_b2,
           dec_W0, dec_b0, dec_W1, dec_b1, dec_W2, dec_b2, codebooks):
    cb_pad = jnp.pad(codebooks, ((0, 0), (0, KP - K_RAW), (0, 0)))
    cbt2 = jnp.transpose(cb_pad, (0, 2, 1)) * jnp.float32(-2.0)
    cb_sc = jnp.pad(codebooks, ((0, 0), (0, 0), (0, 128 - CD)))
    row = lambda v: v.reshape(1, -1)
    full = lambda a: pl.BlockSpec(a.shape, lambda i: (0,) * a.ndim)
    rows_spec = pl.BlockSpec((BM, D_IN), lambda i: (i, 0))
    small_spec = pl.BlockSpec((BM, CD), lambda i: (i, 0))
    gat_spec = pl.BlockSpec((BM, 128), lambda i: (i, 0))
    idx_spec = pl.BlockSpec((BM,), lambda i: (i,))
    loss_spec = pl.BlockSpec((8, 128), lambda i: (0, 0))
    small_shape = jax.ShapeDtypeStruct((BATCH, CD), jnp.float32)
    idx_shape = jax.ShapeDtypeStruct((BATCH,), jnp.int32)
    loss_shape = jax.ShapeDtypeStruct((8, 128), jnp.float32)

    enc_w = [enc_W0, row(enc_b0), enc_W1, row(enc_b1), enc_W2, row(enc_b2)]
    dec_w = [dec_W0, row(dec_b0), dec_W1, row(dec_b1), dec_W2, row(dec_b2)]

    z, i0 = pl.pallas_call(
        _enc_body, grid=(GRID,),
        in_specs=[rows_spec] + [full(w) for w in enc_w] + [full(cbt2[0])],
        out_specs=[small_spec, idx_spec],
        out_shape=[small_shape, idx_shape],
    )(x, *enc_w, cbt2[0])

    q0 = _sc_gather(cb_sc[0], i0)

    r1, i1, lb0 = pl.pallas_call(
        _step_body, grid=(GRID,),
        in_specs=[small_spec, gat_spec, full(cbt2[1])],
        out_specs=[small_spec, idx_spec, loss_spec],
        out_shape=[small_shape, idx_shape, loss_shape],
    )(z, q0, cbt2[1])

    q1 = _sc_gather(cb_sc[1], i1)

    r2, i2, lb1 = pl.pallas_call(
        _step_body, grid=(GRID,),
        in_specs=[small_spec, gat_spec, full(cbt2[2])],
        out_specs=[small_spec, idx_spec, loss_spec],
        out_shape=[small_shape, idx_shape, loss_shape],
    )(r1, q1, cbt2[2])

    q2 = _sc_gather(cb_sc[2], i2)

    recon, lb2 = pl.pallas_call(
        _fin_body, grid=(GRID,),
        in_specs=[small_spec, gat_spec, gat_spec, gat_spec] + [full(w) for w in dec_w],
        out_specs=[rows_spec, loss_spec],
        out_shape=[jax.ShapeDtypeStruct((BATCH, D_IN), jnp.float32), loss_shape],
    )(z, q0, q1, q2, *dec_w)

    indices = jnp.stack([i0, i1, i2], axis=-1)
    commit_loss = jnp.stack([lb0[0, 0], lb1[0, 0], lb2[0, 0]]) * jnp.float32(1.0 / (BATCH * CD))
    return recon, indices, commit_loss


# speedup vs baseline: 1.2238x; 1.1143x over previous
import functools

import jax
import jax.numpy as jnp
from jax import lax
from jax.experimental import pallas as pl
from jax.experimental.pallas import tpu as pltpu
from jax.experimental.pallas import tpu_sc as plsc

BATCH = 4096
D_IN = 4096
K_RAW = 7000
KP = 7040
CD = 32
NQ = 3
BM = 256
GRID = BATCH // BM

_SC_NC = 2
_SC_NS = 16
_NW = _SC_NC * _SC_NS
_BPW = BATCH // _NW


def _vq_argmin(r, cbt2):
    lane1 = lax.broadcasted_iota(jnp.int32, (1, KP), 1)
    lane2 = lax.broadcasted_iota(jnp.int32, (BM, KP), 1)
    n = jnp.sum(cbt2 * cbt2, axis=0, keepdims=True) * jnp.float32(0.25)
    n = jnp.where(lane1 < K_RAW, n, jnp.float32(1e30))
    c = jnp.sum(r * r, axis=-1, keepdims=True)
    m2 = jnp.dot(r, cbt2, preferred_element_type=jnp.float32)
    dist = (c + m2) + n
    return jnp.argmin(dist, axis=-1).astype(jnp.int32)


def _acc_loss(loss_ref, i, s):
    li = lax.broadcasted_iota(jnp.int32, (8, 128), 1)
    part = jnp.where(li == 0, s, 0.0)

    @pl.when(i == 0)
    def _init():
        loss_ref[...] = jnp.zeros_like(loss_ref)

    loss_ref[...] += part


def _enc_body(x_ref, w0, b0, w1, b1, w2, b2, cbt2_ref, z_ref, idx_ref):
    x = x_ref[...]
    h = jnp.maximum(jnp.dot(x, w0[...], preferred_element_type=jnp.float32) + b0[...], 0.0)
    h = jnp.maximum(jnp.dot(h, w1[...], preferred_element_type=jnp.float32) + b1[...], 0.0)
    z = jnp.dot(h, w2[...], preferred_element_type=jnp.float32) + b2[...]
    z_ref[...] = z
    idx_ref[...] = _vq_argmin(z, cbt2_ref[...])


def _step_body(r_ref, qv_ref, cbt2_ref, rn_ref, idx_ref, loss_ref):
    r = r_ref[...] - qv_ref[:, :CD]
    rn_ref[...] = r
    idx_ref[...] = _vq_argmin(r, cbt2_ref[...])
    _acc_loss(loss_ref, pl.program_id(0), jnp.sum(r * r))


def _fin_body(z_ref, q0_ref, q1_ref, q2_ref, dw0, db0, dw1, db1, dw2, db2,
              recon_ref, loss_ref):
    z = z_ref[...]
    q0, q1, q2 = q0_ref[:, :CD], q1_ref[:, :CD], q2_ref[:, :CD]
    r1 = z - q0
    r2 = r1 - q1
    r3 = r2 - q2
    qst0 = z + (q0 - z)
    qst1 = r1 + (q1 - r1)
    qst2 = r2 + (q2 - r2)
    zq = (qst0 + qst1) + qst2
    g = jnp.maximum(jnp.dot(zq, dw0[...], preferred_element_type=jnp.float32) + db0[...], 0.0)
    g = jnp.maximum(jnp.dot(g, dw1[...], preferred_element_type=jnp.float32) + db1[...], 0.0)
    recon_ref[...] = jnp.dot(g, dw2[...], preferred_element_type=jnp.float32) + db2[...]
    _acc_loss(loss_ref, pl.program_id(0), jnp.sum(r3 * r3))


def _sc_gather(table, idx):
    mesh = plsc.VectorSubcoreMesh(core_axis_name="c", subcore_axis_name="s")

    @functools.partial(
        pl.kernel,
        out_type=jax.ShapeDtypeStruct((BATCH, 128), jnp.float32),
        mesh=mesh,
        scratch_types=[
            pltpu.VMEM((_BPW,), jnp.int32),
            pltpu.VMEM((_BPW, 128), jnp.float32),
            pltpu.SemaphoreType.DMA,
        ],
    )
    def k(table_hbm, idx_hbm, out_hbm, idx_v, rows_v, sem):
        wid = lax.axis_index("s") * _SC_NC + lax.axis_index("c")
        base = wid * _BPW
        pltpu.sync_copy(idx_hbm.at[pl.ds(base, _BPW)], idx_v)
        pltpu.async_copy(table_hbm.at[idx_v], rows_v, sem).wait()
        pltpu.sync_copy(rows_v, out_hbm.at[pl.ds(base, _BPW)])

    return k(table, idx)


def kernel(x, enc_W0, enc_b0, enc_W1, enc_b1, enc_W2, enc_b2,
           dec_W0, dec_b0, dec_W1, dec_b1, dec_W2, dec_b2, codebooks):
    cb_pad = jnp.pad(codebooks, ((0, 0), (0, KP - K_RAW), (0, 0)))
    cbt2 = jnp.transpose(cb_pad, (0, 2, 1)) * jnp.float32(-2.0)
    cb_sc = jnp.pad(codebooks, ((0, 0), (0, 0), (0, 128 - CD)))
    row = lambda v: v.reshape(1, -1)
    full = lambda a: pl.BlockSpec(a.shape, lambda i: (0,) * a.ndim)
    rows_spec = pl.BlockSpec((BM, D_IN), lambda i: (i, 0))
    small_spec = pl.BlockSpec((BM, CD), lambda i: (i, 0))
    gat_spec = pl.BlockSpec((BM, 128), lambda i: (i, 0))
    idx_spec = pl.BlockSpec((BM,), lambda i: (i,))
    loss_spec = pl.BlockSpec((8, 128), lambda i: (0, 0))
    small_shape = jax.ShapeDtypeStruct((BATCH, CD), jnp.float32)
    idx_shape = jax.ShapeDtypeStruct((BATCH,), jnp.int32)
    loss_shape = jax.ShapeDtypeStruct((8, 128), jnp.float32)

    enc_w = [enc_W0, row(enc_b0), enc_W1, row(enc_b1), enc_W2, row(enc_b2)]
    dec_w = [dec_W0, row(dec_b0), dec_W1, row(dec_b1), dec_W2, row(dec_b2)]

    z, i0 = pl.pallas_call(
        _enc_body, grid=(GRID,),
        in_specs=[rows_spec] + [full(w) for w in enc_w] + [full(cbt2[0])],
        out_specs=[small_spec, idx_spec],
        out_shape=[small_shape, idx_shape],
    )(x, *enc_w, cbt2[0])

    q0 = _sc_gather(cb_sc[0], i0)

    r1, i1, lb0 = pl.pallas_call(
        _step_body, grid=(GRID,),
        in_specs=[small_spec, gat_spec, full(cbt2[1])],
        out_specs=[small_spec, idx_spec, loss_spec],
        out_shape=[small_shape, idx_shape, loss_shape],
    )(z, q0, cbt2[1])

    q1 = _sc_gather(cb_sc[1], i1)

    r2, i2, lb1 = pl.pallas_call(
        _step_body, grid=(GRID,),
        in_specs=[small_spec, gat_spec, full(cbt2[2])],
        out_specs=[small_spec, idx_spec, loss_spec],
        out_shape=[small_shape, idx_shape, loss_shape],
    )(r1, q1, cbt2[2])

    q2 = _sc_gather(cb_sc[2], i2)

    recon, lb2 = pl.pallas_call(
        _fin_body, grid=(GRID,),
        in_specs=[small_spec, gat_spec, gat_spec, gat_spec] + [full(w) for w in dec_w],
        out_specs=[rows_spec, loss_spec],
        out_shape=[jax.ShapeDtypeStruct((BATCH, D_IN), jnp.float32), loss_shape],
    )(z, q0, q1, q2, *dec_w)

    indices = jnp.stack([i0, i1, i2], axis=-1)
    commit_loss = jnp.stack([lb0[0, 0], lb1[0, 0], lb2[0, 0]]) * jnp.float32(1.0 / (BATCH * CD))
    return recon, indices, commit_loss
